# Initial kernel scaffold; baseline (speedup 1.0000x reference)
import jax
import jax.numpy as jnp
from jax.experimental import pallas as pl


def kernel(h0, h1, h2, assign0, assign1, W_up01, b_up01, W_down10, b_down10, W_up12, b_up12, W_down21, b_down21, g0, bt0, g1, bt1, g2, bt2):
    raise NotImplementedError("write your pallas kernel here")


# speedup vs baseline: 2.8223x; 2.8223x over previous
import functools

import jax
import jax.numpy as jnp
from jax import lax
from jax.experimental import pallas as pl
from jax.experimental.pallas import tpu as pltpu
from jax.experimental.pallas import tpu_sc as plsc

NC = 2
NS = 16
H = 128
HH = H // NC


def _ceil_to(x, m):
    return (x + m - 1) // m * m


def _make_segsum(n_src, n_seg, ch):
    n_chunks = n_src // ch
    assert n_chunks * ch == n_src and ch % 8 == 0
    rounds = -(-n_chunks // NS)
    rpt = _ceil_to(-(-n_seg // NS), 8)
    n_pad = rpt * NS
    zh = rpt // 2
    mesh = plsc.VectorSubcoreMesh(core_axis_name="c", subcore_axis_name="s")

    @functools.partial(
        pl.kernel,
        out_type=[
            jax.ShapeDtypeStruct((NC * n_pad, HH), jnp.float32),
            jax.ShapeDtypeStruct((n_pad, 8), jnp.float32),
        ],
        mesh=mesh,
        compiler_params=pltpu.CompilerParams(use_tc_tiling_on_sc=False),
        scratch_types=[
            pltpu.VMEM_SHARED((n_pad, HH), jnp.float32),
            pltpu.VMEM_SHARED((n_pad, 8), jnp.float32),
            pltpu.VMEM((ch, HH), jnp.float32),
            pltpu.VMEM((ch,), jnp.int32),
            pltpu.VMEM((ch, 8), jnp.float32),
        ],
    )
    def seg(src_hbm, a_hbm, zt_hbm, zc_hbm, one_hbm,
            sums_out, cnt_out, table, ctable, stage, idx, ones):
        c = lax.axis_index("c")
        s = lax.axis_index("s")
        pltpu.sync_copy(zt_hbm, table.at[pl.ds(s * rpt, zh)])
        pltpu.sync_copy(zt_hbm, table.at[pl.ds(s * rpt + zh, zh)])

        @pl.when(c == 0)
        def _():
            pltpu.sync_copy(zc_hbm, ctable.at[pl.ds(s * rpt, rpt)])
            pltpu.sync_copy(one_hbm, ones)

        plsc.subcore_barrier()

        def work(chunk):
            pltpu.sync_copy(
                src_hbm.at[pl.ds(chunk * ch, ch), pl.ds(c * HH, HH)], stage)
            pltpu.sync_copy(a_hbm.at[pl.ds(chunk * ch, ch)], idx)
            pltpu.sync_copy(stage, table.at[idx], add=True)

            @pl.when(c == 0)
            def _():
                pltpu.sync_copy(ones, ctable.at[idx], add=True)

        full_rounds = n_chunks // NS
        tail = n_chunks % NS

        def round_body(i, carry):
            work(i * NS + s)
            return carry

        lax.fori_loop(0, full_rounds, round_body, 0)
        if tail:
            pl.when(s < tail)(lambda: work(full_rounds * NS + s))

        plsc.subcore_barrier()
        pltpu.sync_copy(table.at[pl.ds(s * rpt, rpt)],
                        sums_out.at[pl.ds(c * n_pad + s * rpt, rpt)])

        @pl.when(c == 0)
        def _():
            pltpu.sync_copy(ctable.at[pl.ds(s * rpt, rpt)],
                            cnt_out.at[pl.ds(s * rpt, rpt)])

    def run(src, a):
        zt = jnp.zeros((zh, HH), jnp.float32)
        zc = jnp.zeros((rpt, 8), jnp.float32)
        one = jnp.ones((ch, 8), jnp.float32)
        sums, cnt8 = seg(src, a, zt, zc, one)
        sums_l = sums[:n_seg]
        sums_r = sums[n_pad:n_pad + n_seg]
        cnt = cnt8[:n_seg, 0:1]
        return sums_l, sums_r, cnt

    return run


def _make_gather(n_rows, ch):
    n_chunks = n_rows // ch
    assert n_chunks * ch == n_rows and ch % 8 == 0
    nw = NC * NS
    rounds = -(-n_chunks // nw)
    mesh = plsc.VectorSubcoreMesh(core_axis_name="c", subcore_axis_name="s")

    @functools.partial(
        pl.kernel,
        out_type=jax.ShapeDtypeStruct((n_rows, H), jnp.float32),
        mesh=mesh,
        scratch_types=[
            pltpu.VMEM((ch,), jnp.int32),
            pltpu.VMEM((ch, H), jnp.float32),
            pltpu.SemaphoreType.DMA,
        ],
    )
    def g(tbl_hbm, a_hbm, out_hbm, idx, rows, sem):
        w = lax.axis_index("s") * NC + lax.axis_index("c")
        for i in range(rounds):
            chunk = w + nw * i

            def body(chunk=chunk):
                pltpu.sync_copy(a_hbm.at[pl.ds(chunk * ch, ch)], idx)
                pltpu.async_copy(tbl_hbm.at[idx], rows, sem).wait()
                pltpu.sync_copy(rows, out_hbm.at[pl.ds(chunk * ch, ch)])

            if (i + 1) * nw <= n_chunks:
                body()
            else:
                pl.when(chunk < n_chunks)(body)

    return g


def _make_level_update(n_rows, blk):
    grid = n_rows // blk
    assert grid * blk == n_rows

    def body(sl_ref, sr_ref, cnt_ref, hc_ref, wu_ref, bu_ref, wd_ref, bd_ref,
             g_ref, bt_ref, h_out, t_out):
        sums = jnp.concatenate([sl_ref[...], sr_ref[...]], axis=1)
        mean = sums / jnp.maximum(cnt_ref[...], 1.0)
        msg = lax.dot_general(mean, wu_ref[...], (((1,), (1,)), ((), ())),
                              preferred_element_type=jnp.float32) + bu_ref[...]
        x = hc_ref[...] + msg
        m = jnp.mean(x, axis=-1, keepdims=True)
        v = jnp.mean((x - m) ** 2, axis=-1, keepdims=True)
        y = (x - m) * lax.rsqrt(v + 1e-5) * g_ref[...] + bt_ref[...]
        h_out[...] = y
        t_out[...] = lax.dot_general(y, wd_ref[...], (((1,), (1,)), ((), ())),
                                     preferred_element_type=jnp.float32) + bd_ref[...]

    row_spec = lambda w: pl.BlockSpec((blk, w), lambda i: (i, 0))
    full = pl.BlockSpec((H, H), lambda i: (0, 0))
    vec = pl.BlockSpec((1, H), lambda i: (0, 0))
    f = pl.pallas_call(
        body,
        grid=(grid,),
        in_specs=[row_spec(HH), row_spec(HH), row_spec(1), row_spec(H),
                  full, vec, full, vec, vec, vec],
        out_specs=[row_spec(H), row_spec(H)],
        out_shape=[jax.ShapeDtypeStruct((n_rows, H), jnp.float32),
                   jax.ShapeDtypeStruct((n_rows, H), jnp.float32)],
    )

    def run(sums_l, sums_r, cnt, hc, wu, bu, wd, bd, gg, bt):
        return f(sums_l, sums_r, cnt, hc, wu.reshape(H, H), bu.reshape(1, H),
                 wd.reshape(H, H), bd.reshape(1, H), gg.reshape(1, H),
                 bt.reshape(1, H))

    return run


def _make_res_ln(n_rows, blk):
    grid = n_rows // blk
    assert grid * blk == n_rows

    def body(h_ref, m_ref, g_ref, bt_ref, out_ref):
        x = h_ref[...] + m_ref[...]
        m = jnp.mean(x, axis=-1, keepdims=True)
        v = jnp.mean((x - m) ** 2, axis=-1, keepdims=True)
        out_ref[...] = (x - m) * lax.rsqrt(v + 1e-5) * g_ref[...] + bt_ref[...]

    row_spec = pl.BlockSpec((blk, H), lambda i: (i, 0))
    vec = pl.BlockSpec((1, H), lambda i: (0, 0))
    f = pl.pallas_call(
        body,
        grid=(grid,),
        in_specs=[row_spec, row_spec, vec, vec],
        out_specs=row_spec,
        out_shape=jax.ShapeDtypeStruct((n_rows, H), jnp.float32),
    )

    def run(h, msg, gg, bt):
        return f(h, msg, gg.reshape(1, H), bt.reshape(1, H))

    return run


def kernel(h0, h1, h2, assign0, assign1,
           W_up01, b_up01, W_down10, b_down10,
           W_up12, b_up12, W_down21, b_down21,
           g0, bt0, g1, bt1, g2, bt2):
    n0, n1, n2 = h0.shape[0], h1.shape[0], h2.shape[0]

    segsum01 = _make_segsum(n0, n1, 200)
    segsum12 = _make_segsum(n1, n2, 1000)
    gather10 = _make_gather(n0, 800)
    gather21 = _make_gather(n1, 1000)
    lvl1 = _make_level_update(n1, 1000)
    lvl2 = _make_level_update(n2, n2)
    resln0 = _make_res_ln(n0, 1000)
    resln1 = _make_res_ln(n1, 1000)

    sl, sr, cnt = segsum01(h0, assign0)
    h1a, t1 = lvl1(sl, sr, cnt, h1, W_up01, b_up01, W_down10, b_down10, g1, bt1)
    msg10 = gather10(t1, assign0)
    h0o = resln0(h0, msg10, g0, bt0)
    sl2, sr2, cnt2 = segsum12(h1a, assign1)
    h2o, t2 = lvl2(sl2, sr2, cnt2, h2, W_up12, b_up12, W_down21, b_down21, g2, bt2)
    msg21 = gather21(t2, assign1)
    h1o = resln1(h1a, msg21, g1, bt1)
    return (h0o, h1o, h2o)


# speedup vs baseline: 4.0279x; 1.4272x over previous
import functools

import jax
import jax.numpy as jnp
from jax import lax
from jax.experimental import pallas as pl
from jax.experimental.pallas import tpu as pltpu
from jax.experimental.pallas import tpu_sc as plsc

NC = 2
NS = 16
H = 128
HH = H // NC


def _ceil_to(x, m):
    return (x + m - 1) // m * m


def _make_segsum(n_src, n_seg, ch):
    n_chunks = n_src // ch
    assert n_chunks * ch == n_src and ch % 8 == 0
    rpt = _ceil_to(-(-n_seg // NS), 8)
    n_pad = rpt * NS
    zh = rpt // 2
    mesh = plsc.VectorSubcoreMesh(core_axis_name="c", subcore_axis_name="s")

    @functools.partial(
        pl.kernel,
        out_type=[
            jax.ShapeDtypeStruct((NC * n_seg, HH), jnp.float32),
            jax.ShapeDtypeStruct((n_seg, 8), jnp.float32),
        ],
        mesh=mesh,
        compiler_params=pltpu.CompilerParams(use_tc_tiling_on_sc=False),
        scratch_types=[
            pltpu.VMEM_SHARED((n_pad, HH), jnp.float32),
            pltpu.VMEM_SHARED((n_pad, 8), jnp.float32),
            pltpu.VMEM((ch, HH), jnp.float32),
            pltpu.VMEM((ch,), jnp.int32),
            pltpu.VMEM((ch, 8), jnp.float32),
            pltpu.SemaphoreType.DMA,
            pltpu.SemaphoreType.DMA,
        ],
    )
    def seg(src_hbm, a_hbm, zt_hbm, zc_hbm, one_hbm,
            sums_out, cnt_out, table, ctable, stage, idx, ones, sem_a, sem_b):
        c = lax.axis_index("c")
        s = lax.axis_index("s")
        pltpu.sync_copy(zt_hbm, table.at[pl.ds(s * rpt, zh)])
        pltpu.sync_copy(zt_hbm, table.at[pl.ds(s * rpt + zh, zh)])

        @pl.when(c == 0)
        def _():
            pltpu.sync_copy(zc_hbm, ctable.at[pl.ds(s * rpt, rpt)])
            pltpu.sync_copy(one_hbm, ones)

        plsc.subcore_barrier()

        def work(chunk):
            cp1 = pltpu.async_copy(
                src_hbm.at[pl.ds(chunk * ch, ch), pl.ds(c * HH, HH)], stage,
                sem_a)
            cp2 = pltpu.async_copy(a_hbm.at[pl.ds(chunk * ch, ch)], idx, sem_b)
            cp1.wait()
            cp2.wait()
            cp3 = pltpu.async_copy(stage, table.at[idx], sem_a, add=True)

            @pl.when(c == 0)
            def _():
                pltpu.async_copy(ones, ctable.at[idx], sem_b, add=True).wait()

            cp3.wait()

        full_rounds = n_chunks // NS
        tail = n_chunks % NS

        def round_body(i, carry):
            work(i * NS + s)
            return carry

        lax.fori_loop(0, full_rounds, round_body, 0)
        if tail:
            pl.when(s < tail)(lambda: work(full_rounds * NS + s))

        plsc.subcore_barrier()
        w0 = jnp.minimum(s * rpt, n_seg - rpt)
        pltpu.sync_copy(table.at[pl.ds(w0, rpt)],
                        sums_out.at[pl.ds(c * n_seg + w0, rpt)])

        @pl.when(c == 0)
        def _():
            pltpu.sync_copy(ctable.at[pl.ds(w0, rpt)],
                            cnt_out.at[pl.ds(w0, rpt)])

    def run(src, a):
        zt = jnp.zeros((zh, HH), jnp.float32)
        zc = jnp.zeros((rpt, 8), jnp.float32)
        one = jnp.ones((ch, 8), jnp.float32)
        return seg(src, a, zt, zc, one)

    return run


def _make_gather_pipelined(n_rows, ch):
    n_chunks = n_rows // ch
    assert n_chunks * ch == n_rows and ch % 8 == 0
    nw = NC * NS
    rounds = -(-n_chunks // nw)
    mesh = plsc.VectorSubcoreMesh(core_axis_name="c", subcore_axis_name="s")

    @functools.partial(
        pl.kernel,
        out_type=jax.ShapeDtypeStruct((n_rows, H), jnp.float32),
        mesh=mesh,
        scratch_types=[
            pltpu.VMEM((ch,), jnp.int32),
            pltpu.VMEM((ch,), jnp.int32),
            pltpu.VMEM((ch, H), jnp.float32),
            pltpu.VMEM((ch, H), jnp.float32),
            pltpu.SemaphoreType.DMA,
            pltpu.SemaphoreType.DMA,
            pltpu.SemaphoreType.DMA,
            pltpu.SemaphoreType.DMA,
            pltpu.SemaphoreType.DMA,
            pltpu.SemaphoreType.DMA,
        ],
    )
    def g(tbl_hbm, a_hbm, out_hbm, idx0, idx1, rows0, rows1,
          si0, si1, sg0, sg1, sw0, sw1):
        w = lax.axis_index("s") * NC + lax.axis_index("c")
        idx = [idx0, idx1]
        rows = [rows0, rows1]
        si = [si0, si1]
        sg = [sg0, sg1]
        sw = [sw0, sw1]

        def chunk_of(r):
            return w + nw * r

        def guard(r, f):
            if (r + 1) * nw <= n_chunks:
                f()
            else:
                pl.when(chunk_of(r) < n_chunks)(f)

        def fetch_idx(r):
            b = r % 2
            pltpu.async_copy(a_hbm.at[pl.ds(chunk_of(r) * ch, ch)], idx[b],
                             si[b])

        def wait_idx(r):
            b = r % 2
            pltpu.make_async_copy(a_hbm.at[pl.ds(chunk_of(r) * ch, ch)],
                                  idx[b], si[b]).wait()

        def start_gather(r):
            b = r % 2
            pltpu.async_copy(tbl_hbm.at[idx[b]], rows[b], sg[b])

        def wait_gather(r):
            b = r % 2
            pltpu.make_async_copy(tbl_hbm.at[idx[b]], rows[b], sg[b]).wait()

        def start_wb(r):
            b = r % 2
            pltpu.async_copy(rows[b], out_hbm.at[pl.ds(chunk_of(r) * ch, ch)],
                             sw[b])

        def wait_wb(r):
            b = r % 2
            pltpu.make_async_copy(rows[b],
                                  out_hbm.at[pl.ds(chunk_of(r) * ch, ch)],
                                  sw[b]).wait()

        guard(0, lambda: fetch_idx(0))
        for r in range(rounds):
            guard(r, lambda r=r: wait_idx(r))
            if r >= 2:
                guard(r, lambda r=r: wait_wb(r - 2))
            guard(r, lambda r=r: start_gather(r))
            if r + 1 < rounds:
                guard(r + 1, lambda r=r: fetch_idx(r + 1))
            guard(r, lambda r=r: wait_gather(r))
            guard(r, lambda r=r: start_wb(r))
        for r in range(max(rounds - 2, 0), rounds):
            guard(r, lambda r=r: wait_wb(r))

    return g


def _make_level_update(n_rows, blk):
    grid = n_rows // blk
    assert grid * blk == n_rows

    def body(sl_ref, sr_ref, cnt_ref, hc_ref, wu_ref, bu_ref, wd_ref, bd_ref,
             g_ref, bt_ref, h_out, t_out):
        sums = jnp.concatenate([sl_ref[0], sr_ref[0]], axis=1)
        mean = sums / jnp.maximum(cnt_ref[...][:, :1], 1.0)
        msg = lax.dot_general(mean, wu_ref[...], (((1,), (1,)), ((), ())),
                              preferred_element_type=jnp.float32) + bu_ref[...]
        x = hc_ref[...] + msg
        m = jnp.mean(x, axis=-1, keepdims=True)
        v = jnp.mean((x - m) ** 2, axis=-1, keepdims=True)
        y = (x - m) * lax.rsqrt(v + 1e-5) * g_ref[...] + bt_ref[...]
        h_out[...] = y
        t_out[...] = lax.dot_general(y, wd_ref[...], (((1,), (1,)), ((), ())),
                                     preferred_element_type=jnp.float32) + bd_ref[...]

    full = pl.BlockSpec((H, H), lambda i: (0, 0))
    vec = pl.BlockSpec((1, H), lambda i: (0, 0))
    f = pl.pallas_call(
        body,
        grid=(grid,),
        in_specs=[pl.BlockSpec((1, blk, HH), lambda i: (i, 0, 0)),
                  pl.BlockSpec((1, blk, HH), lambda i: (i + grid, 0, 0)),
                  pl.BlockSpec((blk, 8), lambda i: (i, 0)),
                  pl.BlockSpec((blk, H), lambda i: (i, 0)),
                  full, vec, full, vec, vec, vec],
        out_specs=[pl.BlockSpec((blk, H), lambda i: (i, 0)),
                   pl.BlockSpec((blk, H), lambda i: (i, 0))],
        out_shape=[jax.ShapeDtypeStruct((n_rows, H), jnp.float32),
                   jax.ShapeDtypeStruct((n_rows, H), jnp.float32)],
    )

    def run(sums, cnt8, hc, wu, bu, wd, bd, gg, bt):
        sums = sums.reshape(NC * grid, blk, HH)
        return f(sums, sums, cnt8, hc, wu.reshape(H, H), bu.reshape(1, H),
                 wd.reshape(H, H), bd.reshape(1, H), gg.reshape(1, H),
                 bt.reshape(1, H))

    return run


def _make_res_ln(n_rows, blk):
    grid = n_rows // blk
    assert grid * blk == n_rows

    def body(h_ref, m_ref, g_ref, bt_ref, out_ref):
        x = h_ref[...] + m_ref[...]
        m = jnp.mean(x, axis=-1, keepdims=True)
        v = jnp.mean((x - m) ** 2, axis=-1, keepdims=True)
        out_ref[...] = (x - m) * lax.rsqrt(v + 1e-5) * g_ref[...] + bt_ref[...]

    row_spec = pl.BlockSpec((blk, H), lambda i: (i, 0))
    vec = pl.BlockSpec((1, H), lambda i: (0, 0))
    f = pl.pallas_call(
        body,
        grid=(grid,),
        in_specs=[row_spec, row_spec, vec, vec],
        out_specs=row_spec,
        out_shape=jax.ShapeDtypeStruct((n_rows, H), jnp.float32),
    )

    def run(h, msg, gg, bt):
        return f(h, msg, gg.reshape(1, H), bt.reshape(1, H))

    return run


def kernel(h0, h1, h2, assign0, assign1,
           W_up01, b_up01, W_down10, b_down10,
           W_up12, b_up12, W_down21, b_down21,
           g0, bt0, g1, bt1, g2, bt2):
    n0, n1, n2 = h0.shape[0], h1.shape[0], h2.shape[0]

    segsum01 = _make_segsum(n0, n1, 200)
    segsum12 = _make_segsum(n1, n2, 1000)
    gather10 = _make_gather_pipelined(n0, 400)
    gather21 = _make_gather_pipelined(n1, 1000)
    lvl1 = _make_level_update(n1, 5000)
    lvl2 = _make_level_update(n2, n2)
    resln0 = _make_res_ln(n0, 4000)
    resln1 = _make_res_ln(n1, 5000)

    sums1, cnt1 = segsum01(h0, assign0)
    h1a, t1 = lvl1(sums1, cnt1, h1, W_up01, b_up01, W_down10, b_down10, g1, bt1)
    msg10 = gather10(t1, assign0)
    h0o = resln0(h0, msg10, g0, bt0)
    sums2, cnt2 = segsum12(h1a, assign1)
    h2o, t2 = lvl2(sums2, cnt2, h2, W_up12, b_up12, W_down21, b_down21, g2, bt2)
    msg21 = gather21(t2, assign1)
    h1o = resln1(h1a, msg21, g1, bt1)
    return (h0o, h1o, h2o)


# speedup vs baseline: 4.4018x; 1.0928x over previous
import functools

import jax
import jax.numpy as jnp
from jax import lax
from jax.experimental import pallas as pl
from jax.experimental.pallas import tpu as pltpu
from jax.experimental.pallas import tpu_sc as plsc

NC = 2
NS = 16
H = 128
HH = H // NC


def _ceil_to(x, m):
    return (x + m - 1) // m * m


def _make_segsum(n_src, n_seg, ch):
    n_chunks = n_src // ch
    assert n_chunks * ch == n_src and ch % 8 == 0
    rpt = _ceil_to(-(-n_seg // NS), 16)
    n_pad = rpt * NS
    zh = rpt // 2
    for cw in (96, 80, 64, 48, 32, 16):
        if rpt % cw == 0:
            break
    assert rpt % cw == 0 and cw % 16 == 0
    mesh = plsc.VectorSubcoreMesh(core_axis_name="c", subcore_axis_name="s")

    @functools.partial(
        pl.kernel,
        out_type=jax.ShapeDtypeStruct((n_seg, H), jnp.float32),
        mesh=mesh,
        compiler_params=pltpu.CompilerParams(use_tc_tiling_on_sc=False),
        scratch_types=[
            pltpu.VMEM_SHARED((n_pad, HH), jnp.float32),
            pltpu.VMEM_SHARED((n_pad,), jnp.float32),
            pltpu.VMEM((ch, HH), jnp.float32),
            pltpu.VMEM((ch,), jnp.int32),
            pltpu.VMEM((ch,), jnp.float32),
            pltpu.VMEM((cw, HH), jnp.float32),
            pltpu.VMEM((cw,), jnp.float32),
            pltpu.SemaphoreType.DMA,
            pltpu.SemaphoreType.DMA,
        ],
    )
    def seg(src_hbm, a_hbm, zt_hbm, zc_hbm, one_hbm,
            mean_out, table, ctable, stage, idx, ones, dbuf, cbuf,
            sem_a, sem_b):
        c = lax.axis_index("c")
        s = lax.axis_index("s")
        pltpu.sync_copy(zt_hbm, table.at[pl.ds(s * rpt, zh)])
        pltpu.sync_copy(zt_hbm, table.at[pl.ds(s * rpt + zh, zh)])
        pltpu.sync_copy(zc_hbm, ctable.at[pl.ds(s * rpt, rpt)])
        pltpu.sync_copy(one_hbm, ones)

        plsc.subcore_barrier()

        def work(chunk):
            cp1 = pltpu.async_copy(
                src_hbm.at[pl.ds(chunk * ch, ch), pl.ds(c * HH, HH)], stage,
                sem_a)
            cp2 = pltpu.async_copy(a_hbm.at[pl.ds(chunk * ch, ch)], idx, sem_b)
            cp1.wait()
            cp2.wait()
            cp3 = pltpu.async_copy(stage, table.at[idx], sem_a, add=True)
            cp4 = pltpu.async_copy(ones, ctable.at[idx], sem_b, add=True)
            cp3.wait()
            cp4.wait()

        full_rounds = n_chunks // NS
        tail = n_chunks % NS

        def round_body(i, carry):
            work(i * NS + s)
            return carry

        lax.fori_loop(0, full_rounds, round_body, 0)
        if tail:
            pl.when(s < tail)(lambda: work(full_rounds * NS + s))

        plsc.subcore_barrier()
        d0 = s * rpt

        def wchunk(k, carry):
            r0 = d0 + k * cw
            pltpu.sync_copy(table.at[pl.ds(r0, cw)], dbuf)
            pltpu.sync_copy(ctable.at[pl.ds(r0, cw)], cbuf)

            def grp(q, cc):
                v = jnp.maximum(cbuf[pl.ds(q * 16, 16)], 1.0)
                for j in range(16):
                    cv = v[j]
                    for k4 in range(HH // 16):
                        r = q * 16 + j
                        dbuf[r, pl.ds(k4 * 16, 16)] = \
                            dbuf[r, pl.ds(k4 * 16, 16)] / cv
                return cc

            lax.fori_loop(0, cw // 16, grp, 0)
            pltpu.sync_copy(dbuf, table.at[pl.ds(r0, cw)])
            return carry

        lax.fori_loop(0, rpt // cw, wchunk, 0)
        plsc.subcore_barrier()
        w0 = jnp.minimum(s * rpt, n_seg - rpt)
        pltpu.sync_copy(table.at[pl.ds(w0, rpt)],
                        mean_out.at[pl.ds(w0, rpt), pl.ds(c * HH, HH)])

    def run(src, a):
        zt = jnp.zeros((zh, HH), jnp.float32)
        zc = jnp.zeros((rpt,), jnp.float32)
        one = jnp.ones((ch,), jnp.float32)
        return seg(src, a, zt, zc, one)

    return run


def _make_gather_pipelined(n_rows, ch):
    n_chunks = n_rows // ch
    assert n_chunks * ch == n_rows and ch % 8 == 0
    nw = NC * NS
    rounds = -(-n_chunks // nw)
    mesh = plsc.VectorSubcoreMesh(core_axis_name="c", subcore_axis_name="s")

    @functools.partial(
        pl.kernel,
        out_type=jax.ShapeDtypeStruct((n_rows, H), jnp.float32),
        mesh=mesh,
        scratch_types=[
            pltpu.VMEM((ch,), jnp.int32),
            pltpu.VMEM((ch,), jnp.int32),
            pltpu.VMEM((ch, H), jnp.float32),
            pltpu.VMEM((ch, H), jnp.float32),
            pltpu.SemaphoreType.DMA,
            pltpu.SemaphoreType.DMA,
            pltpu.SemaphoreType.DMA,
            pltpu.SemaphoreType.DMA,
            pltpu.SemaphoreType.DMA,
            pltpu.SemaphoreType.DMA,
        ],
    )
    def g(tbl_hbm, a_hbm, out_hbm, idx0, idx1, rows0, rows1,
          si0, si1, sg0, sg1, sw0, sw1):
        w = lax.axis_index("s") * NC + lax.axis_index("c")
        idx = [idx0, idx1]
        rows = [rows0, rows1]
        si = [si0, si1]
        sg = [sg0, sg1]
        sw = [sw0, sw1]

        def chunk_of(r):
            return w + nw * r

        def guard(r, f):
            if (r + 1) * nw <= n_chunks:
                f()
            else:
                pl.when(chunk_of(r) < n_chunks)(f)

        def fetch_idx(r):
            b = r % 2
            pltpu.async_copy(a_hbm.at[pl.ds(chunk_of(r) * ch, ch)], idx[b],
                             si[b])

        def wait_idx(r):
            b = r % 2
            pltpu.make_async_copy(a_hbm.at[pl.ds(chunk_of(r) * ch, ch)],
                                  idx[b], si[b]).wait()

        def start_gather(r):
            b = r % 2
            pltpu.async_copy(tbl_hbm.at[idx[b]], rows[b], sg[b])

        def wait_gather(r):
            b = r % 2
            pltpu.make_async_copy(tbl_hbm.at[idx[b]], rows[b], sg[b]).wait()

        def start_wb(r):
            b = r % 2
            pltpu.async_copy(rows[b], out_hbm.at[pl.ds(chunk_of(r) * ch, ch)],
                             sw[b])

        def wait_wb(r):
            b = r % 2
            pltpu.make_async_copy(rows[b],
                                  out_hbm.at[pl.ds(chunk_of(r) * ch, ch)],
                                  sw[b]).wait()

        guard(0, lambda: fetch_idx(0))
        for r in range(rounds):
            guard(r, lambda r=r: wait_idx(r))
            if r >= 2:
                guard(r, lambda r=r: wait_wb(r - 2))
            guard(r, lambda r=r: start_gather(r))
            if r + 1 < rounds:
                guard(r + 1, lambda r=r: fetch_idx(r + 1))
            guard(r, lambda r=r: wait_gather(r))
            guard(r, lambda r=r: start_wb(r))
        for r in range(max(rounds - 2, 0), rounds):
            guard(r, lambda r=r: wait_wb(r))

    return g


def _make_level_update(n_rows, blk):
    grid = n_rows // blk
    assert grid * blk == n_rows

    def body(mean_ref, hc_ref, wu_ref, bu_ref, wd_ref, bd_ref,
             g_ref, bt_ref, h_out, t_out):
        msg = lax.dot_general(mean_ref[...], wu_ref[...],
                              (((1,), (1,)), ((), ())),
                              preferred_element_type=jnp.float32) + bu_ref[...]
        x = hc_ref[...] + msg
        m = jnp.mean(x, axis=-1, keepdims=True)
        v = jnp.mean((x - m) ** 2, axis=-1, keepdims=True)
        y = (x - m) * lax.rsqrt(v + 1e-5) * g_ref[...] + bt_ref[...]
        h_out[...] = y
        t_out[...] = lax.dot_general(y, wd_ref[...], (((1,), (1,)), ((), ())),
                                     preferred_element_type=jnp.float32) + bd_ref[...]

    full = pl.BlockSpec((H, H), lambda i: (0, 0))
    vec = pl.BlockSpec((1, H), lambda i: (0, 0))
    f = pl.pallas_call(
        body,
        grid=(grid,),
        in_specs=[pl.BlockSpec((blk, H), lambda i: (i, 0)),
                  pl.BlockSpec((blk, H), lambda i: (i, 0)),
                  full, vec, full, vec, vec, vec],
        out_specs=[pl.BlockSpec((blk, H), lambda i: (i, 0)),
                   pl.BlockSpec((blk, H), lambda i: (i, 0))],
        out_shape=[jax.ShapeDtypeStruct((n_rows, H), jnp.float32),
                   jax.ShapeDtypeStruct((n_rows, H), jnp.float32)],
    )

    def run(mean, hc, wu, bu, wd, bd, gg, bt):
        return f(mean, hc, wu.reshape(H, H), bu.reshape(1, H),
                 wd.reshape(H, H), bd.reshape(1, H), gg.reshape(1, H),
                 bt.reshape(1, H))

    return run


def _make_res_ln(n_rows, blk):
    grid = n_rows // blk
    assert grid * blk == n_rows

    def body(h_ref, m_ref, g_ref, bt_ref, out_ref):
        x = h_ref[...] + m_ref[...]
        m = jnp.mean(x, axis=-1, keepdims=True)
        v = jnp.mean((x - m) ** 2, axis=-1, keepdims=True)
        out_ref[...] = (x - m) * lax.rsqrt(v + 1e-5) * g_ref[...] + bt_ref[...]

    row_spec = pl.BlockSpec((blk, H), lambda i: (i, 0))
    vec = pl.BlockSpec((1, H), lambda i: (0, 0))
    f = pl.pallas_call(
        body,
        grid=(grid,),
        in_specs=[row_spec, row_spec, vec, vec],
        out_specs=row_spec,
        out_shape=jax.ShapeDtypeStruct((n_rows, H), jnp.float32),
    )

    def run(h, msg, gg, bt):
        return f(h, msg, gg.reshape(1, H), bt.reshape(1, H))

    return run


def kernel(h0, h1, h2, assign0, assign1,
           W_up01, b_up01, W_down10, b_down10,
           W_up12, b_up12, W_down21, b_down21,
           g0, bt0, g1, bt1, g2, bt2):
    n0, n1, n2 = h0.shape[0], h1.shape[0], h2.shape[0]

    segsum01 = _make_segsum(n0, n1, 400)
    segsum12 = _make_segsum(n1, n2, 1000)
    gather10 = _make_gather_pipelined(n0, 400)
    gather21 = _make_gather_pipelined(n1, 1000)
    lvl1 = _make_level_update(n1, 5000)
    lvl2 = _make_level_update(n2, n2)
    resln0 = _make_res_ln(n0, 4000)
    resln1 = _make_res_ln(n1, 5000)

    mean1 = segsum01(h0, assign0)
    h1a, t1 = lvl1(mean1, h1, W_up01, b_up01, W_down10, b_down10, g1, bt1)
    msg10 = gather10(t1, assign0)
    h0o = resln0(h0, msg10, g0, bt0)
    mean2 = segsum12(h1a, assign1)
    h2o, t2 = lvl2(mean2, h2, W_up12, b_up12, W_down21, b_down21, g2, bt2)
    msg21 = gather21(t2, assign1)
    h1o = resln1(h1a, msg21, g1, bt1)
    return (h0o, h1o, h2o)


# speedup vs baseline: 4.4759x; 1.0169x over previous
import functools

import jax
import jax.numpy as jnp
from jax import lax
from jax.experimental import pallas as pl
from jax.experimental.pallas import tpu as pltpu
from jax.experimental.pallas import tpu_sc as plsc

NC = 2
NS = 16
H = 128
HH = H // NC


def _ceil_to(x, m):
    return (x + m - 1) // m * m


def _make_segsum(n_src, n_seg, ch):
    n_chunks = n_src // ch
    assert n_chunks * ch == n_src and ch % 8 == 0
    rpt = _ceil_to(-(-n_seg // NS), 16)
    n_pad = rpt * NS
    zh = rpt // 2
    for cw in (112, 96, 80, 64, 48, 32, 16):
        if rpt % cw == 0:
            break
    assert rpt % cw == 0 and cw % 16 == 0
    mesh = plsc.VectorSubcoreMesh(core_axis_name="c", subcore_axis_name="s")

    @functools.partial(
        pl.kernel,
        out_type=jax.ShapeDtypeStruct((n_seg, H), jnp.float32),
        mesh=mesh,
        compiler_params=pltpu.CompilerParams(use_tc_tiling_on_sc=False),
        scratch_types=[
            pltpu.VMEM_SHARED((n_pad, HH), jnp.float32),
            pltpu.VMEM_SHARED((n_pad,), jnp.float32),
            pltpu.VMEM((ch, HH), jnp.float32),
            pltpu.VMEM((ch, HH), jnp.float32),
            pltpu.VMEM((ch,), jnp.int32),
            pltpu.VMEM((ch,), jnp.int32),
            pltpu.VMEM((ch,), jnp.float32),
            pltpu.VMEM((cw, HH), jnp.float32),
            pltpu.VMEM((cw,), jnp.float32),
            pltpu.SemaphoreType.DMA,
            pltpu.SemaphoreType.DMA,
            pltpu.SemaphoreType.DMA,
            pltpu.SemaphoreType.DMA,
        ],
    )
    def seg(src_hbm, a_hbm, zt_hbm, zc_hbm, one_hbm,
            mean_out, table, ctable, stage0, stage1, idx0, idx1, ones,
            dbuf, cbuf, sf0, sf1, ss0, ss1):
        c = lax.axis_index("c")
        s = lax.axis_index("s")
        pltpu.sync_copy(zt_hbm, table.at[pl.ds(s * rpt, zh)])
        pltpu.sync_copy(zt_hbm, table.at[pl.ds(s * rpt + zh, zh)])
        pltpu.sync_copy(zc_hbm, ctable.at[pl.ds(s * rpt, rpt)])
        pltpu.sync_copy(one_hbm, ones)

        plsc.subcore_barrier()

        def base(r):
            return (r * NS + s) * ch

        def fetch_issue(r, st, ix, sem):
            pltpu.async_copy(
                src_hbm.at[pl.ds(base(r), ch), pl.ds(c * HH, HH)], st, sem)
            pltpu.async_copy(a_hbm.at[pl.ds(base(r), ch)], ix, sem)

        def fetch_wait(r, st, ix, sem):
            pltpu.make_async_copy(
                src_hbm.at[pl.ds(base(r), ch), pl.ds(c * HH, HH)], st,
                sem).wait()
            pltpu.make_async_copy(a_hbm.at[pl.ds(base(r), ch)], ix,
                                  sem).wait()

        def scat_issue(st, ix, sem):
            pltpu.async_copy(st, table.at[ix], sem, add=True)
            pltpu.async_copy(ones, ctable.at[ix], sem, add=True)

        def scat_wait(st, ix, sem):
            pltpu.make_async_copy(st, table.at[ix], sem).wait()
            pltpu.make_async_copy(ones, ctable.at[ix], sem).wait()

        def work(chunk):
            cp1 = pltpu.async_copy(
                src_hbm.at[pl.ds(chunk * ch, ch), pl.ds(c * HH, HH)], stage0,
                sf0)
            cp2 = pltpu.async_copy(a_hbm.at[pl.ds(chunk * ch, ch)], idx0, ss0)
            cp1.wait()
            cp2.wait()
            cp3 = pltpu.async_copy(stage0, table.at[idx0], sf0, add=True)
            cp4 = pltpu.async_copy(ones, ctable.at[idx0], ss0, add=True)
            cp3.wait()
            cp4.wait()

        full_rounds = n_chunks // NS
        tail = n_chunks % NS
        npairs = full_rounds // 2

        if npairs > 0:
            def pair(rr, carry):
                a = 2 * rr

                fetch_wait(a, stage0, idx0, sf0)

                @pl.when(rr > 0)
                def _():
                    scat_wait(stage1, idx1, ss1)

                fetch_issue(a + 1, stage1, idx1, sf1)
                scat_issue(stage0, idx0, ss0)
                fetch_wait(a + 1, stage1, idx1, sf1)
                scat_wait(stage0, idx0, ss0)

                @pl.when(rr + 1 < npairs)
                def _():
                    fetch_issue(a + 2, stage0, idx0, sf0)

                scat_issue(stage1, idx1, ss1)
                return carry

            fetch_issue(0, stage0, idx0, sf0)
            lax.fori_loop(0, npairs, pair, 0)
            scat_wait(stage1, idx1, ss1)

        for r in range(2 * npairs, full_rounds):
            work(r * NS + s)
        if tail:
            pl.when(s < tail)(lambda: work(full_rounds * NS + s))

        plsc.subcore_barrier()
        d0 = s * rpt

        def wchunk(k, carry):
            r0 = d0 + k * cw
            pltpu.sync_copy(table.at[pl.ds(r0, cw)], dbuf)
            pltpu.sync_copy(ctable.at[pl.ds(r0, cw)], cbuf)

            def grp(q, cc):
                v = jnp.maximum(cbuf[pl.ds(q * 16, 16)], 1.0)
                for j in range(16):
                    cv = v[j]
                    for k4 in range(HH // 16):
                        r = q * 16 + j
                        dbuf[r, pl.ds(k4 * 16, 16)] = \
                            dbuf[r, pl.ds(k4 * 16, 16)] / cv
                return cc

            lax.fori_loop(0, cw // 16, grp, 0)
            pltpu.sync_copy(dbuf, table.at[pl.ds(r0, cw)])
            return carry

        lax.fori_loop(0, rpt // cw, wchunk, 0)
        plsc.subcore_barrier()
        w0 = jnp.minimum(s * rpt, n_seg - rpt)
        pltpu.sync_copy(table.at[pl.ds(w0, rpt)],
                        mean_out.at[pl.ds(w0, rpt), pl.ds(c * HH, HH)])

    def run(src, a):
        zt = jnp.zeros((zh, HH), jnp.float32)
        zc = jnp.zeros((rpt,), jnp.float32)
        one = jnp.ones((ch,), jnp.float32)
        return seg(src, a, zt, zc, one)

    return run


def _make_gather_pipelined(n_rows, ch):
    n_chunks = n_rows // ch
    assert n_chunks * ch == n_rows and ch % 8 == 0
    nw = NC * NS
    rounds = -(-n_chunks // nw)
    mesh = plsc.VectorSubcoreMesh(core_axis_name="c", subcore_axis_name="s")

    @functools.partial(
        pl.kernel,
        out_type=jax.ShapeDtypeStruct((n_rows, H), jnp.float32),
        mesh=mesh,
        scratch_types=[
            pltpu.VMEM((ch,), jnp.int32),
            pltpu.VMEM((ch,), jnp.int32),
            pltpu.VMEM((ch, H), jnp.float32),
            pltpu.VMEM((ch, H), jnp.float32),
            pltpu.SemaphoreType.DMA,
            pltpu.SemaphoreType.DMA,
            pltpu.SemaphoreType.DMA,
            pltpu.SemaphoreType.DMA,
            pltpu.SemaphoreType.DMA,
            pltpu.SemaphoreType.DMA,
        ],
    )
    def g(tbl_hbm, a_hbm, out_hbm, idx0, idx1, rows0, rows1,
          si0, si1, sg0, sg1, sw0, sw1):
        w = lax.axis_index("s") * NC + lax.axis_index("c")
        idx = [idx0, idx1]
        rows = [rows0, rows1]
        si = [si0, si1]
        sg = [sg0, sg1]
        sw = [sw0, sw1]

        def chunk_of(r):
            return w + nw * r

        def guard(r, f):
            if (r + 1) * nw <= n_chunks:
                f()
            else:
                pl.when(chunk_of(r) < n_chunks)(f)

        def fetch_idx(r):
            b = r % 2
            pltpu.async_copy(a_hbm.at[pl.ds(chunk_of(r) * ch, ch)], idx[b],
                             si[b])

        def wait_idx(r):
            b = r % 2
            pltpu.make_async_copy(a_hbm.at[pl.ds(chunk_of(r) * ch, ch)],
                                  idx[b], si[b]).wait()

        def start_gather(r):
            b = r % 2
            pltpu.async_copy(tbl_hbm.at[idx[b]], rows[b], sg[b])

        def wait_gather(r):
            b = r % 2
            pltpu.make_async_copy(tbl_hbm.at[idx[b]], rows[b], sg[b]).wait()

        def start_wb(r):
            b = r % 2
            pltpu.async_copy(rows[b], out_hbm.at[pl.ds(chunk_of(r) * ch, ch)],
                             sw[b])

        def wait_wb(r):
            b = r % 2
            pltpu.make_async_copy(rows[b],
                                  out_hbm.at[pl.ds(chunk_of(r) * ch, ch)],
                                  sw[b]).wait()

        guard(0, lambda: fetch_idx(0))
        for r in range(rounds):
            guard(r, lambda r=r: wait_idx(r))
            if r >= 2:
                guard(r, lambda r=r: wait_wb(r - 2))
            guard(r, lambda r=r: start_gather(r))
            if r + 1 < rounds:
                guard(r + 1, lambda r=r: fetch_idx(r + 1))
            guard(r, lambda r=r: wait_gather(r))
            guard(r, lambda r=r: start_wb(r))
        for r in range(max(rounds - 2, 0), rounds):
            guard(r, lambda r=r: wait_wb(r))

    return g


def _make_level_update(n_rows, blk):
    grid = n_rows // blk
    assert grid * blk == n_rows

    def body(mean_ref, hc_ref, wu_ref, bu_ref, wd_ref, bd_ref,
             g_ref, bt_ref, h_out, t_out):
        msg = lax.dot_general(mean_ref[...], wu_ref[...],
                              (((1,), (1,)), ((), ())),
                              preferred_element_type=jnp.float32) + bu_ref[...]
        x = hc_ref[...] + msg
        m = jnp.mean(x, axis=-1, keepdims=True)
        v = jnp.mean((x - m) ** 2, axis=-1, keepdims=True)
        y = (x - m) * lax.rsqrt(v + 1e-5) * g_ref[...] + bt_ref[...]
        h_out[...] = y
        t_out[...] = lax.dot_general(y, wd_ref[...], (((1,), (1,)), ((), ())),
                                     preferred_element_type=jnp.float32) + bd_ref[...]

    full = pl.BlockSpec((H, H), lambda i: (0, 0))
    vec = pl.BlockSpec((1, H), lambda i: (0, 0))
    f = pl.pallas_call(
        body,
        grid=(grid,),
        in_specs=[pl.BlockSpec((blk, H), lambda i: (i, 0)),
                  pl.BlockSpec((blk, H), lambda i: (i, 0)),
                  full, vec, full, vec, vec, vec],
        out_specs=[pl.BlockSpec((blk, H), lambda i: (i, 0)),
                   pl.BlockSpec((blk, H), lambda i: (i, 0))],
        out_shape=[jax.ShapeDtypeStruct((n_rows, H), jnp.float32),
                   jax.ShapeDtypeStruct((n_rows, H), jnp.float32)],
    )

    def run(mean, hc, wu, bu, wd, bd, gg, bt):
        return f(mean, hc, wu.reshape(H, H), bu.reshape(1, H),
                 wd.reshape(H, H), bd.reshape(1, H), gg.reshape(1, H),
                 bt.reshape(1, H))

    return run


def _make_res_ln(n_rows, blk):
    grid = n_rows // blk
    assert grid * blk == n_rows

    def body(h_ref, m_ref, g_ref, bt_ref, out_ref):
        x = h_ref[...] + m_ref[...]
        m = jnp.mean(x, axis=-1, keepdims=True)
        v = jnp.mean(x * x, axis=-1, keepdims=True) - m * m
        out_ref[...] = (x - m) * (lax.rsqrt(v + 1e-5) * g_ref[...]) + bt_ref[...]

    row_spec = pl.BlockSpec((blk, H), lambda i: (i, 0))
    vec = pl.BlockSpec((1, H), lambda i: (0, 0))
    f = pl.pallas_call(
        body,
        grid=(grid,),
        in_specs=[row_spec, row_spec, vec, vec],
        out_specs=row_spec,
        out_shape=jax.ShapeDtypeStruct((n_rows, H), jnp.float32),
    )

    def run(h, msg, gg, bt):
        return f(h, msg, gg.reshape(1, H), bt.reshape(1, H))

    return run


def kernel(h0, h1, h2, assign0, assign1,
           W_up01, b_up01, W_down10, b_down10,
           W_up12, b_up12, W_down21, b_down21,
           g0, bt0, g1, bt1, g2, bt2):
    n0, n1, n2 = h0.shape[0], h1.shape[0], h2.shape[0]

    segsum01 = _make_segsum(n0, n1, 160)
    segsum12 = _make_segsum(n1, n2, 200)
    gather10 = _make_gather_pipelined(n0, 400)
    gather21 = _make_gather_pipelined(n1, 1000)
    lvl1 = _make_level_update(n1, 5000)
    lvl2 = _make_level_update(n2, n2)
    resln0 = _make_res_ln(n0, 4000)
    resln1 = _make_res_ln(n1, 5000)

    mean1 = segsum01(h0, assign0)
    h1a, t1 = lvl1(mean1, h1, W_up01, b_up01, W_down10, b_down10, g1, bt1)
    msg10 = gather10(t1, assign0)
    h0o = resln0(h0, msg10, g0, bt0)
    mean2 = segsum12(h1a, assign1)
    h2o, t2 = lvl2(mean2, h2, W_up12, b_up12, W_down21, b_down21, g2, bt2)
    msg21 = gather21(t2, assign1)
    h1o = resln1(h1a, msg21, g1, bt1)
    return (h0o, h1o, h2o)


# speedup vs baseline: 4.4909x; 1.0033x over previous
import functools

import jax
import jax.numpy as jnp
from jax import lax
from jax.experimental import pallas as pl
from jax.experimental.pallas import tpu as pltpu
from jax.experimental.pallas import tpu_sc as plsc

NC = 2
NS = 16
H = 128
HH = H // NC


def _ceil_to(x, m):
    return (x + m - 1) // m * m


def _make_segsum(n_src, n_seg, ch):
    n_chunks = n_src // ch
    assert n_chunks * ch == n_src and ch % 8 == 0
    rpt = _ceil_to(-(-n_seg // NS), 16)
    n_pad = rpt * NS
    zh = rpt // 2
    for cw in (112, 96, 80, 64, 48, 32, 16):
        if rpt % cw == 0:
            break
    assert rpt % cw == 0 and cw % 16 == 0
    mesh = plsc.VectorSubcoreMesh(core_axis_name="c", subcore_axis_name="s")

    @functools.partial(
        pl.kernel,
        out_type=jax.ShapeDtypeStruct((n_seg, H), jnp.float32),
        mesh=mesh,
        compiler_params=pltpu.CompilerParams(use_tc_tiling_on_sc=False),
        scratch_types=[
            pltpu.VMEM_SHARED((n_pad, HH), jnp.float32),
            pltpu.VMEM_SHARED((n_pad,), jnp.float32),
            pltpu.VMEM((ch, HH), jnp.float32),
            pltpu.VMEM((ch, HH), jnp.float32),
            pltpu.VMEM((ch,), jnp.int32),
            pltpu.VMEM((ch,), jnp.int32),
            pltpu.VMEM((ch,), jnp.float32),
            pltpu.VMEM((cw, HH), jnp.float32),
            pltpu.VMEM((cw,), jnp.float32),
            pltpu.SemaphoreType.DMA,
            pltpu.SemaphoreType.DMA,
            pltpu.SemaphoreType.DMA,
            pltpu.SemaphoreType.DMA,
        ],
    )
    def seg(src_hbm, a_hbm, zt_hbm, zc_hbm, one_hbm,
            mean_out, table, ctable, stage0, stage1, idx0, idx1, ones,
            dbuf, cbuf, sf0, sf1, ss0, ss1):
        c = lax.axis_index("c")
        s = lax.axis_index("s")
        pltpu.sync_copy(zt_hbm, table.at[pl.ds(s * rpt, zh)])
        pltpu.sync_copy(zt_hbm, table.at[pl.ds(s * rpt + zh, zh)])
        pltpu.sync_copy(zc_hbm, ctable.at[pl.ds(s * rpt, rpt)])
        pltpu.sync_copy(one_hbm, ones)

        plsc.subcore_barrier()

        def base(r):
            return (r * NS + s) * ch

        def fetch_issue(r, st, ix, sem):
            pltpu.async_copy(
                src_hbm.at[pl.ds(base(r), ch), pl.ds(c * HH, HH)], st, sem)
            pltpu.async_copy(a_hbm.at[pl.ds(base(r), ch)], ix, sem)

        def fetch_wait(r, st, ix, sem):
            pltpu.make_async_copy(
                src_hbm.at[pl.ds(base(r), ch), pl.ds(c * HH, HH)], st,
                sem).wait()
            pltpu.make_async_copy(a_hbm.at[pl.ds(base(r), ch)], ix,
                                  sem).wait()

        def scat_issue(st, ix, sem):
            pltpu.async_copy(st, table.at[ix], sem, add=True)
            pltpu.async_copy(ones, ctable.at[ix], sem, add=True)

        def scat_wait(st, ix, sem):
            pltpu.make_async_copy(st, table.at[ix], sem).wait()
            pltpu.make_async_copy(ones, ctable.at[ix], sem).wait()

        def work(chunk):
            cp1 = pltpu.async_copy(
                src_hbm.at[pl.ds(chunk * ch, ch), pl.ds(c * HH, HH)], stage0,
                sf0)
            cp2 = pltpu.async_copy(a_hbm.at[pl.ds(chunk * ch, ch)], idx0, ss0)
            cp1.wait()
            cp2.wait()
            cp3 = pltpu.async_copy(stage0, table.at[idx0], sf0, add=True)
            cp4 = pltpu.async_copy(ones, ctable.at[idx0], ss0, add=True)
            cp3.wait()
            cp4.wait()

        full_rounds = n_chunks // NS
        tail = n_chunks % NS
        npairs = full_rounds // 2

        if npairs > 0:
            def pair(rr, carry):
                a = 2 * rr

                fetch_wait(a, stage0, idx0, sf0)

                @pl.when(rr > 0)
                def _():
                    scat_wait(stage1, idx1, ss1)

                fetch_issue(a + 1, stage1, idx1, sf1)
                scat_issue(stage0, idx0, ss0)
                fetch_wait(a + 1, stage1, idx1, sf1)
                scat_wait(stage0, idx0, ss0)

                @pl.when(rr + 1 < npairs)
                def _():
                    fetch_issue(a + 2, stage0, idx0, sf0)

                scat_issue(stage1, idx1, ss1)
                return carry

            fetch_issue(0, stage0, idx0, sf0)
            lax.fori_loop(0, npairs, pair, 0)
            scat_wait(stage1, idx1, ss1)

        for r in range(2 * npairs, full_rounds):
            work(r * NS + s)
        if tail:
            pl.when(s < tail)(lambda: work(full_rounds * NS + s))

        plsc.subcore_barrier()
        d0 = s * rpt

        def wchunk(k, carry):
            r0 = d0 + k * cw
            pltpu.sync_copy(table.at[pl.ds(r0, cw)], dbuf)
            pltpu.sync_copy(ctable.at[pl.ds(r0, cw)], cbuf)

            def grp(q, cc):
                v = jnp.maximum(cbuf[pl.ds(q * 16, 16)], 1.0)
                for j in range(16):
                    cv = v[j]
                    for k4 in range(HH // 16):
                        r = q * 16 + j
                        dbuf[r, pl.ds(k4 * 16, 16)] = \
                            dbuf[r, pl.ds(k4 * 16, 16)] / cv
                return cc

            lax.fori_loop(0, cw // 16, grp, 0)
            pltpu.sync_copy(dbuf, table.at[pl.ds(r0, cw)])
            return carry

        lax.fori_loop(0, rpt // cw, wchunk, 0)
        plsc.subcore_barrier()
        w0 = jnp.minimum(s * rpt, n_seg - rpt)
        pltpu.sync_copy(table.at[pl.ds(w0, rpt)],
                        mean_out.at[pl.ds(w0, rpt), pl.ds(c * HH, HH)])

    def run(src, a):
        zt = jnp.zeros((zh, HH), jnp.float32)
        zc = jnp.zeros((rpt,), jnp.float32)
        one = jnp.ones((ch,), jnp.float32)
        return seg(src, a, zt, zc, one)

    return run


def _make_gather_pipelined(n_rows, ch):
    n_chunks = n_rows // ch
    assert n_chunks * ch == n_rows and ch % 8 == 0
    nw = NC * NS
    rounds = -(-n_chunks // nw)
    mesh = plsc.VectorSubcoreMesh(core_axis_name="c", subcore_axis_name="s")

    @functools.partial(
        pl.kernel,
        out_type=jax.ShapeDtypeStruct((n_rows, H), jnp.float32),
        mesh=mesh,
        scratch_types=[
            pltpu.VMEM((ch,), jnp.int32),
            pltpu.VMEM((ch,), jnp.int32),
            pltpu.VMEM((ch, H), jnp.float32),
            pltpu.VMEM((ch, H), jnp.float32),
            pltpu.SemaphoreType.DMA,
            pltpu.SemaphoreType.DMA,
            pltpu.SemaphoreType.DMA,
            pltpu.SemaphoreType.DMA,
            pltpu.SemaphoreType.DMA,
            pltpu.SemaphoreType.DMA,
        ],
    )
    def g(tbl_hbm, a_hbm, out_hbm, idx0, idx1, rows0, rows1,
          si0, si1, sg0, sg1, sw0, sw1):
        w = lax.axis_index("s") * NC + lax.axis_index("c")
        idx = [idx0, idx1]
        rows = [rows0, rows1]
        si = [si0, si1]
        sg = [sg0, sg1]
        sw = [sw0, sw1]

        def chunk_of(r):
            return w + nw * r

        def guard(r, f):
            if (r + 1) * nw <= n_chunks:
                f()
            else:
                pl.when(chunk_of(r) < n_chunks)(f)

        def fetch_idx(r):
            b = r % 2
            pltpu.async_copy(a_hbm.at[pl.ds(chunk_of(r) * ch, ch)], idx[b],
                             si[b])

        def wait_idx(r):
            b = r % 2
            pltpu.make_async_copy(a_hbm.at[pl.ds(chunk_of(r) * ch, ch)],
                                  idx[b], si[b]).wait()

        def start_gather(r):
            b = r % 2
            pltpu.async_copy(tbl_hbm.at[idx[b]], rows[b], sg[b])

        def wait_gather(r):
            b = r % 2
            pltpu.make_async_copy(tbl_hbm.at[idx[b]], rows[b], sg[b]).wait()

        def start_wb(r):
            b = r % 2
            pltpu.async_copy(rows[b], out_hbm.at[pl.ds(chunk_of(r) * ch, ch)],
                             sw[b])

        def wait_wb(r):
            b = r % 2
            pltpu.make_async_copy(rows[b],
                                  out_hbm.at[pl.ds(chunk_of(r) * ch, ch)],
                                  sw[b]).wait()

        guard(0, lambda: fetch_idx(0))
        for r in range(rounds):
            guard(r, lambda r=r: wait_idx(r))
            if r >= 2:
                guard(r, lambda r=r: wait_wb(r - 2))
            guard(r, lambda r=r: start_gather(r))
            if r + 1 < rounds:
                guard(r + 1, lambda r=r: fetch_idx(r + 1))
            guard(r, lambda r=r: wait_gather(r))
            guard(r, lambda r=r: start_wb(r))
        for r in range(max(rounds - 2, 0), rounds):
            guard(r, lambda r=r: wait_wb(r))

    return g


def _make_level_update(n_rows, blk):
    grid = n_rows // blk
    assert grid * blk == n_rows

    def body(mean_ref, hc_ref, wu_ref, bu_ref, wd_ref, bd_ref,
             g_ref, bt_ref, h_out, t_out):
        msg = lax.dot_general(mean_ref[...], wu_ref[...],
                              (((1,), (1,)), ((), ())),
                              preferred_element_type=jnp.float32) + bu_ref[...]
        x = hc_ref[...] + msg
        m = jnp.mean(x, axis=-1, keepdims=True)
        v = jnp.mean((x - m) ** 2, axis=-1, keepdims=True)
        y = (x - m) * lax.rsqrt(v + 1e-5) * g_ref[...] + bt_ref[...]
        h_out[...] = y
        t_out[...] = lax.dot_general(y, wd_ref[...], (((1,), (1,)), ((), ())),
                                     preferred_element_type=jnp.float32) + bd_ref[...]

    full = pl.BlockSpec((H, H), lambda i: (0, 0))
    vec = pl.BlockSpec((1, H), lambda i: (0, 0))
    f = pl.pallas_call(
        body,
        grid=(grid,),
        in_specs=[pl.BlockSpec((blk, H), lambda i: (i, 0)),
                  pl.BlockSpec((blk, H), lambda i: (i, 0)),
                  full, vec, full, vec, vec, vec],
        out_specs=[pl.BlockSpec((blk, H), lambda i: (i, 0)),
                   pl.BlockSpec((blk, H), lambda i: (i, 0))],
        out_shape=[jax.ShapeDtypeStruct((n_rows, H), jnp.float32),
                   jax.ShapeDtypeStruct((n_rows, H), jnp.float32)],
    )

    def run(mean, hc, wu, bu, wd, bd, gg, bt):
        return f(mean, hc, wu.reshape(H, H), bu.reshape(1, H),
                 wd.reshape(H, H), bd.reshape(1, H), gg.reshape(1, H),
                 bt.reshape(1, H))

    return run


def _make_res_ln(n_rows, blk):
    grid = n_rows // blk
    assert grid * blk == n_rows

    def body(h_ref, m_ref, g_ref, bt_ref, out_ref):
        x = h_ref[...] + m_ref[...]
        m = jnp.mean(x, axis=-1, keepdims=True)
        v = jnp.mean(x * x, axis=-1, keepdims=True) - m * m
        out_ref[...] = (x - m) * (lax.rsqrt(v + 1e-5) * g_ref[...]) + bt_ref[...]

    row_spec = pl.BlockSpec((blk, H), lambda i: (i, 0))
    vec = pl.BlockSpec((1, H), lambda i: (0, 0))
    f = pl.pallas_call(
        body,
        grid=(grid,),
        in_specs=[row_spec, row_spec, vec, vec],
        out_specs=row_spec,
        out_shape=jax.ShapeDtypeStruct((n_rows, H), jnp.float32),
    )

    def run(h, msg, gg, bt):
        return f(h, msg, gg.reshape(1, H), bt.reshape(1, H))

    return run


def kernel(h0, h1, h2, assign0, assign1,
           W_up01, b_up01, W_down10, b_down10,
           W_up12, b_up12, W_down21, b_down21,
           g0, bt0, g1, bt1, g2, bt2):
    n0, n1, n2 = h0.shape[0], h1.shape[0], h2.shape[0]

    segsum01 = _make_segsum(n0, n1, 160)
    segsum12 = _make_segsum(n1, n2, 200)
    gather10 = _make_gather_pipelined(n0, 400)
    gather21 = _make_gather_pipelined(n1, 1000)
    lvl1 = _make_level_update(n1, 5000)
    lvl2 = _make_level_update(n2, n2)
    resln0 = _make_res_ln(n0, 10000)
    resln1 = _make_res_ln(n1, 5000)

    mean1 = segsum01(h0, assign0)
    h1a, t1 = lvl1(mean1, h1, W_up01, b_up01, W_down10, b_down10, g1, bt1)
    msg10 = gather10(t1, assign0)
    h0o = resln0(h0, msg10, g0, bt0)
    h1a_b, _ = lax.optimization_barrier((h1a, msg10))
    mean2 = segsum12(h1a_b, assign1)
    h2o, t2 = lvl2(mean2, h2, W_up12, b_up12, W_down21, b_down21, g2, bt2)
    msg21 = gather21(t2, assign1)
    h1o = resln1(h1a, msg21, g1, bt1)
    return (h0o, h1o, h2o)


# speedup vs baseline: 4.6672x; 1.0393x over previous
import functools

import jax
import jax.numpy as jnp
from jax import lax
from jax.experimental import pallas as pl
from jax.experimental.pallas import tpu as pltpu
from jax.experimental.pallas import tpu_sc as plsc

NC = 2
NS = 16
H = 128
HH = H // NC


def _ceil_to(x, m):
    return (x + m - 1) // m * m


def _make_segsum(n_src, n_seg, ch):
    n_chunks = n_src // ch
    assert n_chunks * ch == n_src and ch % 8 == 0
    rpt = _ceil_to(-(-n_seg // NS), 16)
    n_pad = rpt * NS
    zh = rpt // 2
    for cw in (112, 96, 80, 64, 48, 32, 16):
        if rpt % cw == 0:
            break
    assert rpt % cw == 0 and cw % 16 == 0
    mesh = plsc.VectorSubcoreMesh(core_axis_name="c", subcore_axis_name="s")

    @functools.partial(
        pl.kernel,
        out_type=jax.ShapeDtypeStruct((n_seg, H), jnp.float32),
        mesh=mesh,
        compiler_params=pltpu.CompilerParams(use_tc_tiling_on_sc=False),
        scratch_types=[
            pltpu.VMEM_SHARED((n_pad, HH), jnp.float32),
            pltpu.VMEM_SHARED((n_pad,), jnp.float32),
            pltpu.VMEM((ch, HH), jnp.float32),
            pltpu.VMEM((ch, HH), jnp.float32),
            pltpu.VMEM((ch,), jnp.int32),
            pltpu.VMEM((ch,), jnp.int32),
            pltpu.VMEM((ch,), jnp.float32),
            pltpu.VMEM((cw, HH), jnp.float32),
            pltpu.VMEM((cw,), jnp.float32),
            pltpu.SemaphoreType.DMA,
            pltpu.SemaphoreType.DMA,
            pltpu.SemaphoreType.DMA,
            pltpu.SemaphoreType.DMA,
        ],
    )
    def seg(src_hbm, a_hbm, zt_hbm, zc_hbm, one_hbm,
            mean_out, table, ctable, stage0, stage1, idx0, idx1, ones,
            dbuf, cbuf, sf0, sf1, ss0, ss1):
        c = lax.axis_index("c")
        s = lax.axis_index("s")
        pltpu.sync_copy(zt_hbm, table.at[pl.ds(s * rpt, zh)])
        pltpu.sync_copy(zt_hbm, table.at[pl.ds(s * rpt + zh, zh)])
        pltpu.sync_copy(zc_hbm, ctable.at[pl.ds(s * rpt, rpt)])
        pltpu.sync_copy(one_hbm, ones)

        plsc.subcore_barrier()

        def base(r):
            return (r * NS + s) * ch

        def fetch_issue(r, st, ix, sem):
            pltpu.async_copy(
                src_hbm.at[pl.ds(base(r), ch), pl.ds(c * HH, HH)], st, sem)
            pltpu.async_copy(a_hbm.at[pl.ds(base(r), ch)], ix, sem)

        def fetch_wait(r, st, ix, sem):
            pltpu.make_async_copy(
                src_hbm.at[pl.ds(base(r), ch), pl.ds(c * HH, HH)], st,
                sem).wait()
            pltpu.make_async_copy(a_hbm.at[pl.ds(base(r), ch)], ix,
                                  sem).wait()

        def scat_issue(st, ix, sem):
            pltpu.async_copy(st, table.at[ix], sem, add=True)
            pltpu.async_copy(ones, ctable.at[ix], sem, add=True)

        def scat_wait(st, ix, sem):
            pltpu.make_async_copy(st, table.at[ix], sem).wait()
            pltpu.make_async_copy(ones, ctable.at[ix], sem).wait()

        def work(chunk):
            cp1 = pltpu.async_copy(
                src_hbm.at[pl.ds(chunk * ch, ch), pl.ds(c * HH, HH)], stage0,
                sf0)
            cp2 = pltpu.async_copy(a_hbm.at[pl.ds(chunk * ch, ch)], idx0, ss0)
            cp1.wait()
            cp2.wait()
            cp3 = pltpu.async_copy(stage0, table.at[idx0], sf0, add=True)
            cp4 = pltpu.async_copy(ones, ctable.at[idx0], ss0, add=True)
            cp3.wait()
            cp4.wait()

        full_rounds = n_chunks // NS
        tail = n_chunks % NS
        npairs = full_rounds // 2

        if npairs > 0:
            def pair(rr, carry):
                a = 2 * rr

                fetch_wait(a, stage0, idx0, sf0)

                @pl.when(rr > 0)
                def _():
                    scat_wait(stage1, idx1, ss1)

                fetch_issue(a + 1, stage1, idx1, sf1)
                scat_issue(stage0, idx0, ss0)
                fetch_wait(a + 1, stage1, idx1, sf1)
                scat_wait(stage0, idx0, ss0)

                @pl.when(rr + 1 < npairs)
                def _():
                    fetch_issue(a + 2, stage0, idx0, sf0)

                scat_issue(stage1, idx1, ss1)
                return carry

            fetch_issue(0, stage0, idx0, sf0)
            lax.fori_loop(0, npairs, pair, 0)
            scat_wait(stage1, idx1, ss1)

        for r in range(2 * npairs, full_rounds):
            work(r * NS + s)
        if tail:
            pl.when(s < tail)(lambda: work(full_rounds * NS + s))

        plsc.subcore_barrier()
        d0 = s * rpt

        def wchunk(k, carry):
            r0 = d0 + k * cw
            pltpu.sync_copy(table.at[pl.ds(r0, cw)], dbuf)
            pltpu.sync_copy(ctable.at[pl.ds(r0, cw)], cbuf)

            def grp(q, cc):
                v = jnp.maximum(cbuf[pl.ds(q * 16, 16)], 1.0)
                for j in range(16):
                    cv = v[j]
                    for k4 in range(HH // 16):
                        r = q * 16 + j
                        dbuf[r, pl.ds(k4 * 16, 16)] = \
                            dbuf[r, pl.ds(k4 * 16, 16)] / cv
                return cc

            lax.fori_loop(0, cw // 16, grp, 0)
            pltpu.sync_copy(dbuf, table.at[pl.ds(r0, cw)])
            return carry

        lax.fori_loop(0, rpt // cw, wchunk, 0)
        plsc.subcore_barrier()
        w0 = jnp.minimum(s * rpt, n_seg - rpt)
        pltpu.sync_copy(table.at[pl.ds(w0, rpt)],
                        mean_out.at[pl.ds(w0, rpt), pl.ds(c * HH, HH)])

    def run(src, a):
        zt = jnp.zeros((zh, HH), jnp.float32)
        zc = jnp.zeros((rpt,), jnp.float32)
        one = jnp.ones((ch,), jnp.float32)
        return seg(src, a, zt, zc, one)

    return run


def _make_gather_pipelined(n_rows, ch):
    n_chunks = n_rows // ch
    assert n_chunks * ch == n_rows and ch % 8 == 0
    nw = NC * NS
    rounds = -(-n_chunks // nw)
    mesh = plsc.VectorSubcoreMesh(core_axis_name="c", subcore_axis_name="s")

    @functools.partial(
        pl.kernel,
        out_type=jax.ShapeDtypeStruct((n_rows, H), jnp.float32),
        mesh=mesh,
        scratch_types=[
            pltpu.VMEM((ch,), jnp.int32),
            pltpu.VMEM((ch,), jnp.int32),
            pltpu.VMEM((ch, H), jnp.float32),
            pltpu.VMEM((ch, H), jnp.float32),
            pltpu.SemaphoreType.DMA,
            pltpu.SemaphoreType.DMA,
            pltpu.SemaphoreType.DMA,
            pltpu.SemaphoreType.DMA,
            pltpu.SemaphoreType.DMA,
            pltpu.SemaphoreType.DMA,
        ],
    )
    def g(tbl_hbm, a_hbm, out_hbm, idx0, idx1, rows0, rows1,
          si0, si1, sg0, sg1, sw0, sw1):
        w = lax.axis_index("s") * NC + lax.axis_index("c")
        idx = [idx0, idx1]
        rows = [rows0, rows1]
        si = [si0, si1]
        sg = [sg0, sg1]
        sw = [sw0, sw1]

        def chunk_of(r):
            return w + nw * r

        def guard(r, f):
            if (r + 1) * nw <= n_chunks:
                f()
            else:
                pl.when(chunk_of(r) < n_chunks)(f)

        def fetch_idx(r):
            b = r % 2
            pltpu.async_copy(a_hbm.at[pl.ds(chunk_of(r) * ch, ch)], idx[b],
                             si[b])

        def wait_idx(r):
            b = r % 2
            pltpu.make_async_copy(a_hbm.at[pl.ds(chunk_of(r) * ch, ch)],
                                  idx[b], si[b]).wait()

        def start_gather(r):
            b = r % 2
            pltpu.async_copy(tbl_hbm.at[idx[b]], rows[b], sg[b])

        def wait_gather(r):
            b = r % 2
            pltpu.make_async_copy(tbl_hbm.at[idx[b]], rows[b], sg[b]).wait()

        def start_wb(r):
            b = r % 2
            pltpu.async_copy(rows[b], out_hbm.at[pl.ds(chunk_of(r) * ch, ch)],
                             sw[b])

        def wait_wb(r):
            b = r % 2
            pltpu.make_async_copy(rows[b],
                                  out_hbm.at[pl.ds(chunk_of(r) * ch, ch)],
                                  sw[b]).wait()

        guard(0, lambda: fetch_idx(0))
        for r in range(rounds):
            guard(r, lambda r=r: wait_idx(r))
            if r >= 2:
                guard(r, lambda r=r: wait_wb(r - 2))
            guard(r, lambda r=r: start_gather(r))
            if r + 1 < rounds:
                guard(r + 1, lambda r=r: fetch_idx(r + 1))
            guard(r, lambda r=r: wait_gather(r))
            guard(r, lambda r=r: start_wb(r))
        for r in range(max(rounds - 2, 0), rounds):
            guard(r, lambda r=r: wait_wb(r))

    return g


def _make_level_update(n_rows, blk):
    grid = n_rows // blk
    assert grid * blk == n_rows

    def body(mean_ref, hc_ref, wu_ref, bu_ref, wd_ref, bd_ref,
             g_ref, bt_ref, h_out, t_out):
        msg = lax.dot_general(mean_ref[...], wu_ref[...],
                              (((1,), (1,)), ((), ())),
                              preferred_element_type=jnp.float32) + bu_ref[...]
        x = hc_ref[...] + msg
        m = jnp.mean(x, axis=-1, keepdims=True)
        v = jnp.mean((x - m) ** 2, axis=-1, keepdims=True)
        y = (x - m) * lax.rsqrt(v + 1e-5) * g_ref[...] + bt_ref[...]
        h_out[...] = y
        t_out[...] = lax.dot_general(y, wd_ref[...], (((1,), (1,)), ((), ())),
                                     preferred_element_type=jnp.float32) + bd_ref[...]

    full = pl.BlockSpec((H, H), lambda i: (0, 0))
    vec = pl.BlockSpec((1, H), lambda i: (0, 0))
    f = pl.pallas_call(
        body,
        grid=(grid,),
        in_specs=[pl.BlockSpec((blk, H), lambda i: (i, 0)),
                  pl.BlockSpec((blk, H), lambda i: (i, 0)),
                  full, vec, full, vec, vec, vec],
        out_specs=[pl.BlockSpec((blk, H), lambda i: (i, 0)),
                   pl.BlockSpec((blk, H), lambda i: (i, 0))],
        out_shape=[jax.ShapeDtypeStruct((n_rows, H), jnp.float32),
                   jax.ShapeDtypeStruct((n_rows, H), jnp.float32)],
    )

    def run(mean, hc, wu, bu, wd, bd, gg, bt):
        return f(mean, hc, wu.reshape(H, H), bu.reshape(1, H),
                 wd.reshape(H, H), bd.reshape(1, H), gg.reshape(1, H),
                 bt.reshape(1, H))

    return run


def _make_res_ln(n_rows, blk):
    grid = n_rows // blk
    assert grid * blk == n_rows

    def body(h_ref, m_ref, g_ref, bt_ref, out_ref):
        x = h_ref[...] + m_ref[...]
        m = jnp.mean(x, axis=-1, keepdims=True)
        v = jnp.mean(x * x, axis=-1, keepdims=True) - m * m
        out_ref[...] = (x - m) * (lax.rsqrt(v + 1e-5) * g_ref[...]) + bt_ref[...]

    row_spec = pl.BlockSpec((blk, H), lambda i: (i, 0))
    vec = pl.BlockSpec((1, H), lambda i: (0, 0))
    f = pl.pallas_call(
        body,
        grid=(grid,),
        in_specs=[row_spec, row_spec, vec, vec],
        out_specs=row_spec,
        out_shape=jax.ShapeDtypeStruct((n_rows, H), jnp.float32),
    )

    def run(h, msg, gg, bt):
        return f(h, msg, gg.reshape(1, H), bt.reshape(1, H))

    return run


def kernel(h0, h1, h2, assign0, assign1,
           W_up01, b_up01, W_down10, b_down10,
           W_up12, b_up12, W_down21, b_down21,
           g0, bt0, g1, bt1, g2, bt2):
    n0, n1, n2 = h0.shape[0], h1.shape[0], h2.shape[0]

    segsum01 = _make_segsum(n0, n1, 160)
    segsum12 = _make_segsum(n1, n2, 200)
    gather10 = _make_gather_pipelined(n0, 400)
    gather21 = _make_gather_pipelined(n1, 200)
    lvl1 = _make_level_update(n1, 5000)
    lvl2 = _make_level_update(n2, n2)
    resln0 = _make_res_ln(n0, 10000)
    resln1 = _make_res_ln(n1, 5000)

    mean1 = segsum01(h0, assign0)
    h1a, t1 = lvl1(mean1, h1, W_up01, b_up01, W_down10, b_down10, g1, bt1)
    msg10 = gather10(t1, assign0)
    h0o = resln0(h0, msg10, g0, bt0)
    h1a_b, _ = lax.optimization_barrier((h1a, msg10))
    mean2 = segsum12(h1a_b, assign1)
    mean2_b, _ = lax.optimization_barrier((mean2, h0o))
    h2o, t2 = lvl2(mean2_b, h2, W_up12, b_up12, W_down21, b_down21, g2, bt2)
    msg21 = gather21(t2, assign1)
    h1o = resln1(h1a, msg21, g1, bt1)
    return (h0o, h1o, h2o)


# speedup vs baseline: 4.7885x; 1.0260x over previous
import functools

import jax
import jax.numpy as jnp
from jax import lax
from jax.experimental import pallas as pl
from jax.experimental.pallas import tpu as pltpu
from jax.experimental.pallas import tpu_sc as plsc

NC = 2
NS = 16
H = 128
HH = H // NC


def _ceil_to(x, m):
    return (x + m - 1) // m * m


def _make_segsum(n_src, n_seg, ch):
    n_chunks = n_src // ch
    assert n_chunks * ch == n_src and ch % 8 == 0
    rpt = _ceil_to(-(-n_seg // NS), 16)
    n_pad = rpt * NS
    zh = rpt // 2
    for cw in (112, 96, 80, 64, 48, 32, 16):
        if rpt % cw == 0:
            break
    assert rpt % cw == 0 and cw % 16 == 0
    mesh = plsc.VectorSubcoreMesh(core_axis_name="c", subcore_axis_name="s")

    @functools.partial(
        pl.kernel,
        out_type=jax.ShapeDtypeStruct((n_seg, H), jnp.float32),
        mesh=mesh,
        compiler_params=pltpu.CompilerParams(use_tc_tiling_on_sc=False),
        scratch_types=[
            pltpu.VMEM_SHARED((n_pad, HH), jnp.float32),
            pltpu.VMEM_SHARED((n_pad,), jnp.float32),
            pltpu.VMEM((ch, HH), jnp.float32),
            pltpu.VMEM((ch, HH), jnp.float32),
            pltpu.VMEM((ch,), jnp.int32),
            pltpu.VMEM((ch,), jnp.int32),
            pltpu.VMEM((ch,), jnp.float32),
            pltpu.VMEM((cw, HH), jnp.float32),
            pltpu.VMEM((cw,), jnp.float32),
            pltpu.SemaphoreType.DMA,
            pltpu.SemaphoreType.DMA,
            pltpu.SemaphoreType.DMA,
            pltpu.SemaphoreType.DMA,
        ],
    )
    def seg(src_hbm, a_hbm, zt_hbm, zc_hbm, one_hbm,
            mean_out, table, ctable, stage0, stage1, idx0, idx1, ones,
            dbuf, cbuf, sf0, sf1, ss0, ss1):
        c = lax.axis_index("c")
        s = lax.axis_index("s")
        pltpu.sync_copy(zt_hbm, table.at[pl.ds(s * rpt, zh)])
        pltpu.sync_copy(zt_hbm, table.at[pl.ds(s * rpt + zh, zh)])
        pltpu.sync_copy(zc_hbm, ctable.at[pl.ds(s * rpt, rpt)])
        pltpu.sync_copy(one_hbm, ones)

        plsc.subcore_barrier()

        def base(r):
            return (r * NS + s) * ch

        def fetch_issue(r, st, ix, sem):
            pltpu.async_copy(
                src_hbm.at[pl.ds(base(r), ch), pl.ds(c * HH, HH)], st, sem)
            pltpu.async_copy(a_hbm.at[pl.ds(base(r), ch)], ix, sem)

        def fetch_wait(r, st, ix, sem):
            pltpu.make_async_copy(
                src_hbm.at[pl.ds(base(r), ch), pl.ds(c * HH, HH)], st,
                sem).wait()
            pltpu.make_async_copy(a_hbm.at[pl.ds(base(r), ch)], ix,
                                  sem).wait()

        def scat_issue(st, ix, sem):
            pltpu.async_copy(st, table.at[ix], sem, add=True)
            pltpu.async_copy(ones, ctable.at[ix], sem, add=True)

        def scat_wait(st, ix, sem):
            pltpu.make_async_copy(st, table.at[ix], sem).wait()
            pltpu.make_async_copy(ones, ctable.at[ix], sem).wait()

        def work(chunk):
            cp1 = pltpu.async_copy(
                src_hbm.at[pl.ds(chunk * ch, ch), pl.ds(c * HH, HH)], stage0,
                sf0)
            cp2 = pltpu.async_copy(a_hbm.at[pl.ds(chunk * ch, ch)], idx0, ss0)
            cp1.wait()
            cp2.wait()
            cp3 = pltpu.async_copy(stage0, table.at[idx0], sf0, add=True)
            cp4 = pltpu.async_copy(ones, ctable.at[idx0], ss0, add=True)
            cp3.wait()
            cp4.wait()

        full_rounds = n_chunks // NS
        tail = n_chunks % NS
        npairs = full_rounds // 2

        if npairs > 0:
            def pair(rr, carry):
                a = 2 * rr

                fetch_wait(a, stage0, idx0, sf0)

                @pl.when(rr > 0)
                def _():
                    scat_wait(stage1, idx1, ss1)

                fetch_issue(a + 1, stage1, idx1, sf1)
                scat_issue(stage0, idx0, ss0)
                fetch_wait(a + 1, stage1, idx1, sf1)
                scat_wait(stage0, idx0, ss0)

                @pl.when(rr + 1 < npairs)
                def _():
                    fetch_issue(a + 2, stage0, idx0, sf0)

                scat_issue(stage1, idx1, ss1)
                return carry

            fetch_issue(0, stage0, idx0, sf0)
            lax.fori_loop(0, npairs, pair, 0)
            scat_wait(stage1, idx1, ss1)

        for r in range(2 * npairs, full_rounds):
            work(r * NS + s)
        if tail:
            pl.when(s < tail)(lambda: work(full_rounds * NS + s))

        plsc.subcore_barrier()
        d0 = s * rpt

        def wchunk(k, carry):
            r0 = d0 + k * cw
            pltpu.sync_copy(table.at[pl.ds(r0, cw)], dbuf)
            pltpu.sync_copy(ctable.at[pl.ds(r0, cw)], cbuf)

            def grp(q, cc):
                v = jnp.maximum(cbuf[pl.ds(q * 16, 16)], 1.0)
                for j in range(16):
                    cv = v[j]
                    for k4 in range(HH // 16):
                        r = q * 16 + j
                        dbuf[r, pl.ds(k4 * 16, 16)] = \
                            dbuf[r, pl.ds(k4 * 16, 16)] / cv
                return cc

            lax.fori_loop(0, cw // 16, grp, 0)
            pltpu.sync_copy(dbuf, table.at[pl.ds(r0, cw)])
            return carry

        lax.fori_loop(0, rpt // cw, wchunk, 0)
        plsc.subcore_barrier()
        w0 = jnp.minimum(s * rpt, n_seg - rpt)
        pltpu.sync_copy(table.at[pl.ds(w0, rpt)],
                        mean_out.at[pl.ds(w0, rpt), pl.ds(c * HH, HH)])

    def run(src, a):
        zt = jnp.zeros((zh, HH), jnp.float32)
        zc = jnp.zeros((rpt,), jnp.float32)
        one = jnp.ones((ch,), jnp.float32)
        return seg(src, a, zt, zc, one)

    return run


def _make_gather_pipelined(n_rows, ch):
    n_chunks = n_rows // ch
    assert n_chunks * ch == n_rows and ch % 8 == 0
    nw = NC * NS
    rounds = -(-n_chunks // nw)
    mesh = plsc.VectorSubcoreMesh(core_axis_name="c", subcore_axis_name="s")

    @functools.partial(
        pl.kernel,
        out_type=jax.ShapeDtypeStruct((n_rows, H), jnp.float32),
        mesh=mesh,
        scratch_types=[
            pltpu.VMEM((ch,), jnp.int32),
            pltpu.VMEM((ch,), jnp.int32),
            pltpu.VMEM((ch, H), jnp.float32),
            pltpu.VMEM((ch, H), jnp.float32),
            pltpu.SemaphoreType.DMA,
            pltpu.SemaphoreType.DMA,
            pltpu.SemaphoreType.DMA,
            pltpu.SemaphoreType.DMA,
            pltpu.SemaphoreType.DMA,
            pltpu.SemaphoreType.DMA,
        ],
    )
    def g(tbl_hbm, a_hbm, out_hbm, idx0, idx1, rows0, rows1,
          si0, si1, sg0, sg1, sw0, sw1):
        w = lax.axis_index("s") * NC + lax.axis_index("c")
        idx = [idx0, idx1]
        rows = [rows0, rows1]
        si = [si0, si1]
        sg = [sg0, sg1]
        sw = [sw0, sw1]

        def chunk_of(r):
            return w + nw * r

        def guard(r, f):
            if (r + 1) * nw <= n_chunks:
                f()
            else:
                pl.when(chunk_of(r) < n_chunks)(f)

        def fetch_idx(r):
            b = r % 2
            pltpu.async_copy(a_hbm.at[pl.ds(chunk_of(r) * ch, ch)], idx[b],
                             si[b])

        def wait_idx(r):
            b = r % 2
            pltpu.make_async_copy(a_hbm.at[pl.ds(chunk_of(r) * ch, ch)],
                                  idx[b], si[b]).wait()

        def start_gather(r):
            b = r % 2
            pltpu.async_copy(tbl_hbm.at[idx[b]], rows[b], sg[b])

        def wait_gather(r):
            b = r % 2
            pltpu.make_async_copy(tbl_hbm.at[idx[b]], rows[b], sg[b]).wait()

        def start_wb(r):
            b = r % 2
            pltpu.async_copy(rows[b], out_hbm.at[pl.ds(chunk_of(r) * ch, ch)],
                             sw[b])

        def wait_wb(r):
            b = r % 2
            pltpu.make_async_copy(rows[b],
                                  out_hbm.at[pl.ds(chunk_of(r) * ch, ch)],
                                  sw[b]).wait()

        guard(0, lambda: fetch_idx(0))
        for r in range(rounds):
            guard(r, lambda r=r: wait_idx(r))
            if r >= 2:
                guard(r, lambda r=r: wait_wb(r - 2))
            guard(r, lambda r=r: start_gather(r))
            if r >= 1:
                guard(r - 1, lambda r=r: wait_gather(r - 1))
                guard(r - 1, lambda r=r: start_wb(r - 1))
            if r + 1 < rounds:
                guard(r + 1, lambda r=r: fetch_idx(r + 1))
        guard(rounds - 1, lambda: wait_gather(rounds - 1))
        guard(rounds - 1, lambda: start_wb(rounds - 1))
        for r in range(max(rounds - 2, 0), rounds):
            guard(r, lambda r=r: wait_wb(r))

    return g


def _make_level_update(n_rows, blk):
    grid = n_rows // blk
    assert grid * blk == n_rows

    def body(mean_ref, hc_ref, wu_ref, bu_ref, wd_ref, bd_ref,
             g_ref, bt_ref, h_out, t_out):
        msg = lax.dot_general(mean_ref[...], wu_ref[...],
                              (((1,), (1,)), ((), ())),
                              preferred_element_type=jnp.float32) + bu_ref[...]
        x = hc_ref[...] + msg
        m = jnp.mean(x, axis=-1, keepdims=True)
        v = jnp.mean((x - m) ** 2, axis=-1, keepdims=True)
        y = (x - m) * lax.rsqrt(v + 1e-5) * g_ref[...] + bt_ref[...]
        h_out[...] = y
        t_out[...] = lax.dot_general(y, wd_ref[...], (((1,), (1,)), ((), ())),
                                     preferred_element_type=jnp.float32) + bd_ref[...]

    full = pl.BlockSpec((H, H), lambda i: (0, 0))
    vec = pl.BlockSpec((1, H), lambda i: (0, 0))
    f = pl.pallas_call(
        body,
        grid=(grid,),
        in_specs=[pl.BlockSpec((blk, H), lambda i: (i, 0)),
                  pl.BlockSpec((blk, H), lambda i: (i, 0)),
                  full, vec, full, vec, vec, vec],
        out_specs=[pl.BlockSpec((blk, H), lambda i: (i, 0)),
                   pl.BlockSpec((blk, H), lambda i: (i, 0))],
        out_shape=[jax.ShapeDtypeStruct((n_rows, H), jnp.float32),
                   jax.ShapeDtypeStruct((n_rows, H), jnp.float32)],
    )

    def run(mean, hc, wu, bu, wd, bd, gg, bt):
        return f(mean, hc, wu.reshape(H, H), bu.reshape(1, H),
                 wd.reshape(H, H), bd.reshape(1, H), gg.reshape(1, H),
                 bt.reshape(1, H))

    return run


def _make_res_ln(n_rows, blk):
    grid = n_rows // blk
    assert grid * blk == n_rows

    def body(h_ref, m_ref, g_ref, bt_ref, out_ref):
        x = h_ref[...] + m_ref[...]
        m = jnp.mean(x, axis=-1, keepdims=True)
        v = jnp.mean(x * x, axis=-1, keepdims=True) - m * m
        out_ref[...] = (x - m) * (lax.rsqrt(v + 1e-5) * g_ref[...]) + bt_ref[...]

    row_spec = pl.BlockSpec((blk, H), lambda i: (i, 0))
    vec = pl.BlockSpec((1, H), lambda i: (0, 0))
    f = pl.pallas_call(
        body,
        grid=(grid,),
        in_specs=[row_spec, row_spec, vec, vec],
        out_specs=row_spec,
        out_shape=jax.ShapeDtypeStruct((n_rows, H), jnp.float32),
    )

    def run(h, msg, gg, bt):
        return f(h, msg, gg.reshape(1, H), bt.reshape(1, H))

    return run


def kernel(h0, h1, h2, assign0, assign1,
           W_up01, b_up01, W_down10, b_down10,
           W_up12, b_up12, W_down21, b_down21,
           g0, bt0, g1, bt1, g2, bt2):
    n0, n1, n2 = h0.shape[0], h1.shape[0], h2.shape[0]

    segsum01 = _make_segsum(n0, n1, 160)
    segsum12 = _make_segsum(n1, n2, 200)
    gather10 = _make_gather_pipelined(n0, 400)
    gather21 = _make_gather_pipelined(n1, 200)
    lvl1 = _make_level_update(n1, 5000)
    lvl2 = _make_level_update(n2, n2)
    resln0 = _make_res_ln(n0, 10000)
    resln1 = _make_res_ln(n1, 5000)

    mean1 = segsum01(h0, assign0)
    h1a, t1 = lvl1(mean1, h1, W_up01, b_up01, W_down10, b_down10, g1, bt1)
    msg10 = gather10(t1, assign0)
    h0o = resln0(h0, msg10, g0, bt0)
    h1a_b, _ = lax.optimization_barrier((h1a, msg10))
    mean2 = segsum12(h1a_b, assign1)
    mean2_b, _ = lax.optimization_barrier((mean2, h0o))
    h2o, t2 = lvl2(mean2_b, h2, W_up12, b_up12, W_down21, b_down21, g2, bt2)
    msg21 = gather21(t2, assign1)
    h1o = resln1(h1a, msg21, g1, bt1)
    return (h0o, h1o, h2o)
